# Initial kernel scaffold; baseline (speedup 1.0000x reference)
import jax
import jax.numpy as jnp
from jax import lax
from jax.experimental import pallas as pl
from jax.experimental.pallas import tpu as pltpu
from jax.experimental.pallas import tpu_sc as plsc

N = 10000
E = 320000
H = 128
HALF = 64
NC = 2
NS = 16
EB = 80
ES = E // NS
NCHUNK = ES // EB
NR = N // NS
RB = 125
NRCHUNK = NR // RB

_f32 = jnp.float32


def _sc_edge_body(xin_hbm, ea_hbm, src_hbm, dst_hbm, t_hbm, u_hbm, aggr_hbm,
                  acc_sh, srcb, dstb, idxb, xsb, eab, valb, nbuf, obuf, tb, ub):
    c = lax.axis_index("c")
    s = lax.axis_index("s")

    pltpu.sync_copy(t_hbm, tb)
    pltpu.sync_copy(u_hbm, ub)
    tv = tb[...]
    uv = ub[...]

    zero16 = jnp.zeros((16,), _f32)

    def _zrow(r, _):
        for q in range(2 * HALF // 16):
            nbuf[r, pl.ds(16 * q, 16)] = zero16
        return 0

    lax.fori_loop(0, RB, _zrow, 0)
    for k in range(NRCHUNK):
        pltpu.sync_copy(nbuf, acc_sh.at[pl.ds(s * NR + k * RB, RB)])
    plsc.subcore_barrier()

    base_e = s * ES
    cN = c * N
    cE = c * E

    def _chunk(j, _):
        e0 = base_e + j * EB
        pltpu.sync_copy(src_hbm.at[pl.ds(e0, EB)], srcb)
        pltpu.sync_copy(dst_hbm.at[pl.ds(e0, EB)], dstb)
        for q in range(EB // 16):
            idxb[pl.ds(16 * q, 16)] = srcb[pl.ds(16 * q, 16)] + cN
        pltpu.sync_copy(xin_hbm.at[idxb], xsb)
        pltpu.sync_copy(ea_hbm.at[pl.ds(cE + e0, EB)], eab)

        def _edge(e, _):
            for q in range(HALF // 16):
                xq = xsb[e, pl.ds(16 * q, 16)]
                aq = eab[e, pl.ds(16 * q, 16)]
                msg = jnp.maximum(xq + aq, 0.0) + 1e-7
                p = jnp.exp(msg * tv - uv)
                valb[e, pl.ds(16 * q, 16)] = msg * p
                valb[e, pl.ds(HALF + 16 * q, 16)] = p
            return 0

        lax.fori_loop(0, EB, _edge, 0)
        pltpu.sync_copy(valb, acc_sh.at[dstb], add=True)
        return 0

    lax.fori_loop(0, NCHUNK, _chunk, 0)
    plsc.subcore_barrier()

    for k in range(NRCHUNK):
        r0 = s * NR + k * RB
        pltpu.sync_copy(acc_sh.at[pl.ds(r0, RB)], nbuf)

        def _row(r, _):
            for q in range(HALF // 16):
                num = nbuf[r, pl.ds(16 * q, 16)]
                den = nbuf[r, pl.ds(HALF + 16 * q, 16)]
                obuf[r, pl.ds(16 * q, 16)] = jnp.where(den > 0.0, num / den, 0.0)
            return 0

        lax.fori_loop(0, RB, _row, 0)
        pltpu.sync_copy(obuf, aggr_hbm.at[pl.ds(cN + r0, RB)])


_sc_edge = pl.kernel(
    _sc_edge_body,
    out_type=jax.ShapeDtypeStruct((NC * N, HALF), _f32),
    mesh=plsc.VectorSubcoreMesh(core_axis_name="c", subcore_axis_name="s"),
    scratch_types=[
        pltpu.VMEM_SHARED((N, 2 * HALF), _f32),
        pltpu.VMEM((EB,), jnp.int32),
        pltpu.VMEM((EB,), jnp.int32),
        pltpu.VMEM((EB,), jnp.int32),
        pltpu.VMEM((EB, HALF), _f32),
        pltpu.VMEM((EB, HALF), _f32),
        pltpu.VMEM((EB, 2 * HALF), _f32),
        pltpu.VMEM((RB, 2 * HALF), _f32),
        pltpu.VMEM((RB, HALF), _f32),
        pltpu.VMEM((16,), _f32),
        pltpu.VMEM((16,), _f32),
    ],
)


def _k_edge_enc(ea_ref, w_ref, b_ref, out_ref, mx_ref):
    i = pl.program_id(0)
    ea = jnp.dot(ea_ref[...], w_ref[...], preferred_element_type=_f32) + b_ref[...]
    out_ref[0] = ea[:, :HALF]
    out_ref[1] = ea[:, HALF:]
    m = jnp.max(ea)

    @pl.when(i == 0)
    def _():
        mx_ref[0, 0] = m

    @pl.when(i > 0)
    def _():
        mx_ref[0, 0] = jnp.maximum(mx_ref[0, 0], m)


def _k_node_enc(x_ref, w_ref, b_ref, out_ref, mx_ref):
    i = pl.program_id(0)
    y = jnp.dot(x_ref[...], w_ref[...], preferred_element_type=_f32) + b_ref[...]
    out_ref[0] = y[:, :HALF]
    out_ref[1] = y[:, HALF:]
    m = jnp.max(y)

    @pl.when(i == 0)
    def _():
        mx_ref[0, 0] = m

    @pl.when(i > 0)
    def _():
        mx_ref[0, 0] = jnp.maximum(mx_ref[0, 0], m)


def _ln_rows(x, g, b):
    mu = jnp.mean(x, axis=-1, keepdims=True)
    xc = x - mu
    var = jnp.mean(xc * xc, axis=-1, keepdims=True)
    return xc * jax.lax.rsqrt(var + 1e-5) * g + b


def _k_norm_act(x_ref, g_ref, b_ref, out_ref, mx_ref):
    i = pl.program_id(0)
    y = jnp.maximum(_ln_rows(x_ref[...], g_ref[...], b_ref[...]), 0.0)
    out_ref[0] = y[:, :HALF]
    out_ref[1] = y[:, HALF:]
    m = jnp.max(y)

    @pl.when(i == 0)
    def _():
        mx_ref[0, 0] = m

    @pl.when(i > 0)
    def _():
        mx_ref[0, 0] = jnp.maximum(mx_ref[0, 0], m)


def _k_mlp(xin_ref, aggr_ref, xprev_ref, w1_ref, b1_ref, g1_ref, be1_ref,
           w2_ref, b2_ref, out_ref):
    h = jnp.concatenate(
        [xin_ref[0] + aggr_ref[0], xin_ref[1] + aggr_ref[1]], axis=-1)
    z = jnp.dot(h, w1_ref[...], preferred_element_type=_f32) + b1_ref[...]
    z = jnp.maximum(_ln_rows(z, g1_ref[...], be1_ref[...]), 0.0)
    z = jnp.dot(z, w2_ref[...], preferred_element_type=_f32) + b2_ref[...]
    out_ref[...] = xprev_ref[...] + z


def _k_final(x_ref, g_ref, b_ref, w_ref, bo_ref, out_ref):
    y = jnp.maximum(_ln_rows(x_ref[...], g_ref[...], b_ref[...]), 0.0)
    out_ref[...] = jnp.dot(y, w_ref[...], preferred_element_type=_f32) + bo_ref[...]


def _full(shape):
    nd = len(shape)
    return pl.BlockSpec(shape, lambda i, _nd=nd: (0,) * _nd)


def _split_max_call(body, x, w, b, bn):
    n = x.shape[0]
    grid = n // bn
    return pl.pallas_call(
        body,
        grid=(grid,),
        in_specs=[
            pl.BlockSpec((bn, x.shape[1]), lambda i: (i, 0)),
            _full(w.shape),
            _full(b.shape),
        ],
        out_specs=[
            pl.BlockSpec((2, bn, HALF), lambda i: (0, i, 0)),
            pl.BlockSpec(memory_space=pltpu.SMEM),
        ],
        out_shape=[
            jax.ShapeDtypeStruct((2, n, HALF), _f32),
            jax.ShapeDtypeStruct((1, 1), _f32),
        ],
    )(x, w, b)


BN = 2000
BE_ENC = 4000


def kernel(x, edge_index, edge_attr, params):
    src = edge_index[0]
    dst = edge_index[1]

    We, be = params['edge_enc']
    ea2, mx_ea = _split_max_call(_k_edge_enc, edge_attr, We, be.reshape(1, H),
                                 BE_ENC)
    ea_flat = ea2.reshape(NC * E, HALF)

    Wn, bn_ = params['node_enc']
    xin2, mx_x = _split_max_call(_k_node_enc, x, Wn, bn_.reshape(1, H), BN)

    mlp_call = pl.pallas_call(
        _k_mlp,
        grid=(N // BN,),
        in_specs=[
            pl.BlockSpec((2, BN, HALF), lambda i: (0, i, 0)),
            pl.BlockSpec((2, BN, HALF), lambda i: (0, i, 0)),
            pl.BlockSpec((BN, H), lambda i: (i, 0)),
            _full((H, 2 * H)), _full((1, 2 * H)), _full((1, 2 * H)),
            _full((1, 2 * H)), _full((2 * H, H)), _full((1, H)),
        ],
        out_specs=pl.BlockSpec((BN, H), lambda i: (i, 0)),
        out_shape=jax.ShapeDtypeStruct((N, H), _f32),
    )

    x_run = jnp.zeros((N, H), _f32)
    for li, lp in enumerate(params['layers']):
        if li > 0:
            g, bb = lp['norm']
            xin2, mx_x = _split_max_call(
                _k_norm_act, x_run, g.reshape(1, H), bb.reshape(1, H), BN)
        t = lp['t']
        u = t * (jnp.maximum(mx_x[0, 0] + mx_ea[0, 0], 0.0) + 1e-7)
        t16 = jnp.broadcast_to(t.astype(_f32), (16,))
        u16 = jnp.broadcast_to(u.astype(_f32), (16,))
        aggr_flat = _sc_edge(xin2.reshape(NC * N, HALF), ea_flat, src, dst,
                             t16, u16)
        aggr2 = aggr_flat.reshape(NC, N, HALF)
        W1, b1 = lp['mlp_w1']
        g1, be1 = lp['mlp_ln']
        W2, b2 = lp['mlp_w2']
        x_run = mlp_call(xin2, aggr2, x_run, W1, b1.reshape(1, 2 * H),
                         g1.reshape(1, 2 * H), be1.reshape(1, 2 * H),
                         W2, b2.reshape(1, H))

    g0, b0 = params['layers'][0]['norm']
    Wo, bo = params['lin_out']
    out = pl.pallas_call(
        _k_final,
        grid=(N // BN,),
        in_specs=[
            pl.BlockSpec((BN, H), lambda i: (i, 0)),
            _full((1, H)), _full((1, H)), _full((H, H)), _full((1, H)),
        ],
        out_specs=pl.BlockSpec((BN, H), lambda i: (i, 0)),
        out_shape=jax.ShapeDtypeStruct((N, H), _f32),
    )(x_run, g0.reshape(1, H), b0.reshape(1, H), Wo, bo.reshape(1, H))
    return out


# speedup vs baseline: 2.0258x; 2.0258x over previous
import jax
import jax.numpy as jnp
from jax import lax
from jax.experimental import pallas as pl
from jax.experimental.pallas import tpu as pltpu
from jax.experimental.pallas import tpu_sc as plsc

N = 10000
E = 320000
H = 128
HALF = 64
NC = 2
NS = 16
EB = 80
ES = E // NS
NCHUNK = ES // EB
ZB = 80
NZCHUNK = N // ZB
NZROUND = (NZCHUNK + NS - 1) // NS
RB = 200
NRCHUNK = N // RB
NRROUND = (NRCHUNK + NS - 1) // NS

_f32 = jnp.float32


def _sc_edge_body(xin_hbm, ea_hbm, src_hbm, dst_hbm, t_hbm, u_hbm, acc_hbm,
                  acc_sh, srcb, dstb, xsb, eab, valb, tb, ub):
    c = lax.axis_index("c")
    s = lax.axis_index("s")

    pltpu.sync_copy(t_hbm, tb)
    pltpu.sync_copy(u_hbm, ub)
    tv = tb[...]
    uv = ub[...]

    zero16 = jnp.zeros((16,), _f32)

    def _zrow(r, _):
        for q in range(2 * HALF // 16):
            valb[r, pl.ds(16 * q, 16)] = zero16
        return 0

    lax.fori_loop(0, ZB, _zrow, 0)
    for k in range(NZROUND):
        cid = s + NS * k

        @pl.when(cid < NZCHUNK)
        def _():
            pltpu.sync_copy(valb, acc_sh.at[pl.ds(cid * ZB, ZB)])
    plsc.subcore_barrier()

    base_e = s * ES
    c64 = c * HALF

    def _chunk(j, _):
        e0 = base_e + j * EB
        pltpu.sync_copy(src_hbm.at[pl.ds(e0, EB)], srcb)
        pltpu.sync_copy(dst_hbm.at[pl.ds(e0, EB)], dstb)
        pltpu.sync_copy(xin_hbm.at[srcb], xsb)
        pltpu.sync_copy(ea_hbm.at[pl.ds(e0, EB)], eab)

        def _edge(e, _):
            for q in range(HALF // 16):
                xq = xsb[e, pl.ds(c64 + 16 * q, 16)]
                aq = eab[e, pl.ds(c64 + 16 * q, 16)]
                msg = jnp.maximum(xq + aq, 0.0) + 1e-7
                p = jnp.exp(msg * tv - uv)
                valb[e, pl.ds(16 * q, 16)] = msg * p
                valb[e, pl.ds(HALF + 16 * q, 16)] = p
            return 0

        lax.fori_loop(0, EB, _edge, 0)
        pltpu.sync_copy(valb, acc_sh.at[dstb], add=True)
        return 0

    lax.fori_loop(0, NCHUNK, _chunk, 0)
    plsc.subcore_barrier()

    cN = c * N
    for k in range(NRROUND):
        cid = s + NS * k

        @pl.when(cid < NRCHUNK)
        def _():
            r0 = cid * RB
            pltpu.sync_copy(acc_sh.at[pl.ds(r0, RB)],
                            acc_hbm.at[pl.ds(cN + r0, RB)])


_sc_edge = pl.kernel(
    _sc_edge_body,
    out_type=jax.ShapeDtypeStruct((NC * N, 2 * HALF), _f32),
    mesh=plsc.VectorSubcoreMesh(core_axis_name="c", subcore_axis_name="s"),
    scratch_types=[
        pltpu.VMEM_SHARED((N, 2 * HALF), _f32),
        pltpu.VMEM((EB,), jnp.int32),
        pltpu.VMEM((EB,), jnp.int32),
        pltpu.VMEM((EB, H), _f32),
        pltpu.VMEM((EB, H), _f32),
        pltpu.VMEM((EB, 2 * HALF), _f32),
        pltpu.VMEM((16,), _f32),
        pltpu.VMEM((16,), _f32),
    ],
)


def _acc_max(mx_ref, m):
    i = pl.program_id(0)

    @pl.when(i == 0)
    def _():
        mx_ref[0, 0] = m

    @pl.when(i > 0)
    def _():
        mx_ref[0, 0] = jnp.maximum(mx_ref[0, 0], m)


def _k_edge_enc(ea_ref, w_ref, b_ref, out_ref, mx_ref):
    ea = jnp.dot(ea_ref[...], w_ref[...], preferred_element_type=_f32) + b_ref[...]
    out_ref[...] = ea
    _acc_max(mx_ref, jnp.max(ea))


def _k_node_enc(x_ref, w_ref, b_ref, out_ref, mx_ref):
    y = jnp.dot(x_ref[...], w_ref[...], preferred_element_type=_f32) + b_ref[...]
    out_ref[...] = y
    _acc_max(mx_ref, jnp.max(y))


def _ln_rows(x, g, b):
    mu = jnp.mean(x, axis=-1, keepdims=True)
    xc = x - mu
    var = jnp.mean(xc * xc, axis=-1, keepdims=True)
    return xc * jax.lax.rsqrt(var + 1e-5) * g + b


def _k_norm_act(x_ref, g_ref, b_ref, out_ref, mx_ref):
    y = jnp.maximum(_ln_rows(x_ref[...], g_ref[...], b_ref[...]), 0.0)
    out_ref[...] = y
    _acc_max(mx_ref, jnp.max(y))


def _k_mlp(xin_ref, acc_ref, xprev_ref, w1_ref, b1_ref, g1_ref, be1_ref,
           w2_ref, b2_ref, out_ref):
    a0 = acc_ref[0]
    a1 = acc_ref[1]
    lo = jnp.where(a0[:, HALF:] > 0.0, a0[:, :HALF] / a0[:, HALF:], 0.0)
    hi = jnp.where(a1[:, HALF:] > 0.0, a1[:, :HALF] / a1[:, HALF:], 0.0)
    h = xin_ref[...] + jnp.concatenate([lo, hi], axis=-1)
    z = jnp.dot(h, w1_ref[...], preferred_element_type=_f32) + b1_ref[...]
    z = jnp.maximum(_ln_rows(z, g1_ref[...], be1_ref[...]), 0.0)
    z = jnp.dot(z, w2_ref[...], preferred_element_type=_f32) + b2_ref[...]
    out_ref[...] = xprev_ref[...] + z


def _k_final(x_ref, g_ref, b_ref, w_ref, bo_ref, out_ref):
    y = jnp.maximum(_ln_rows(x_ref[...], g_ref[...], b_ref[...]), 0.0)
    out_ref[...] = jnp.dot(y, w_ref[...], preferred_element_type=_f32) + bo_ref[...]


def _full(shape):
    nd = len(shape)
    return pl.BlockSpec(shape, lambda i, _nd=nd: (0,) * _nd)


def _enc_max_call(body, x, w, b, bn):
    n = x.shape[0]
    return pl.pallas_call(
        body,
        grid=(n // bn,),
        in_specs=[
            pl.BlockSpec((bn, x.shape[1]), lambda i: (i, 0)),
            _full(w.shape),
            _full(b.shape),
        ],
        out_specs=[
            pl.BlockSpec((bn, H), lambda i: (i, 0)),
            pl.BlockSpec(memory_space=pltpu.SMEM),
        ],
        out_shape=[
            jax.ShapeDtypeStruct((n, H), _f32),
            jax.ShapeDtypeStruct((1, 1), _f32),
        ],
    )(x, w, b)


BN = 2000
BE_ENC = 4000


def kernel(x, edge_index, edge_attr, params):
    src = edge_index[0]
    dst = edge_index[1]

    We, be = params['edge_enc']
    ea, mx_ea = _enc_max_call(_k_edge_enc, edge_attr, We, be.reshape(1, H),
                              BE_ENC)

    Wn, bn_ = params['node_enc']
    xin, mx_x = _enc_max_call(_k_node_enc, x, Wn, bn_.reshape(1, H), BN)

    mlp_call = pl.pallas_call(
        _k_mlp,
        grid=(N // BN,),
        in_specs=[
            pl.BlockSpec((BN, H), lambda i: (i, 0)),
            pl.BlockSpec((2, BN, H), lambda i: (0, i, 0)),
            pl.BlockSpec((BN, H), lambda i: (i, 0)),
            _full((H, 2 * H)), _full((1, 2 * H)), _full((1, 2 * H)),
            _full((1, 2 * H)), _full((2 * H, H)), _full((1, H)),
        ],
        out_specs=pl.BlockSpec((BN, H), lambda i: (i, 0)),
        out_shape=jax.ShapeDtypeStruct((N, H), _f32),
    )

    x_run = jnp.zeros((N, H), _f32)
    for li, lp in enumerate(params['layers']):
        if li > 0:
            g, bb = lp['norm']
            xin, mx_x = _enc_max_call(
                _k_norm_act, x_run, g.reshape(1, H), bb.reshape(1, H), BN)
        t = lp['t']
        u = t * (jnp.maximum(mx_x[0, 0] + mx_ea[0, 0], 0.0) + 1e-7)
        t16 = jnp.broadcast_to(t.astype(_f32), (16,))
        u16 = jnp.broadcast_to(u.astype(_f32), (16,))
        acc_flat = _sc_edge(xin, ea, src, dst, t16, u16)
        acc2 = acc_flat.reshape(NC, N, 2 * HALF)
        W1, b1 = lp['mlp_w1']
        g1, be1 = lp['mlp_ln']
        W2, b2 = lp['mlp_w2']
        x_run = mlp_call(xin, acc2, x_run, W1, b1.reshape(1, 2 * H),
                         g1.reshape(1, 2 * H), be1.reshape(1, 2 * H),
                         W2, b2.reshape(1, H))

    g0, b0 = params['layers'][0]['norm']
    Wo, bo = params['lin_out']
    out = pl.pallas_call(
        _k_final,
        grid=(N // BN,),
        in_specs=[
            pl.BlockSpec((BN, H), lambda i: (i, 0)),
            _full((1, H)), _full((1, H)), _full((H, H)), _full((1, H)),
        ],
        out_specs=pl.BlockSpec((BN, H), lambda i: (i, 0)),
        out_shape=jax.ShapeDtypeStruct((N, H), _f32),
    )(x_run, g0.reshape(1, H), b0.reshape(1, H), Wo, bo.reshape(1, H))
    return out


# speedup vs baseline: 7.3960x; 3.6510x over previous
import jax
import jax.numpy as jnp
from jax import lax
from jax.experimental import pallas as pl
from jax.experimental.pallas import tpu as pltpu
from jax.experimental.pallas import tpu_sc as plsc

N = 10000
E = 320000
H = 128
HALF = 64
NC = 2
NS = 16
EB = 40
ES = E // NS
NCHUNK = ES // EB
NPAIR = NCHUNK // 2
ZB = 40
NZCHUNK = N // ZB
NZROUND = (NZCHUNK + NS - 1) // NS
RB = 200
NRCHUNK = N // RB
NRROUND = (NRCHUNK + NS - 1) // NS

_f32 = jnp.float32


def _sc_edge_body(xin_hbm, ea_hbm, src_hbm, dst_hbm, t_hbm, u_hbm, acc_hbm,
                  acc_sh, srcb0, srcb1, dstb0, dstb1, xsb0, xsb1, eab0, eab1,
                  valb, tb, ub, sidx0, sidx1, sdat0, sdat1):
    srcb = (srcb0, srcb1)
    dstb = (dstb0, dstb1)
    xsb = (xsb0, xsb1)
    eab = (eab0, eab1)
    sidx = (sidx0, sidx1)
    sdat = (sdat0, sdat1)
    c = lax.axis_index("c")
    s = lax.axis_index("s")

    pltpu.sync_copy(t_hbm, tb)
    pltpu.sync_copy(u_hbm, ub)
    tv = tb[...]
    uv = ub[...]

    zero16 = jnp.zeros((16,), _f32)

    def _zrow(r, _):
        for q in range(2 * HALF // 16):
            valb[r, pl.ds(16 * q, 16)] = zero16
        return 0

    lax.fori_loop(0, ZB, _zrow, 0)
    for k in range(NZROUND):
        cid = s + NS * k

        @pl.when(cid < NZCHUNK)
        def _():
            pltpu.sync_copy(valb, acc_sh.at[pl.ds(cid * ZB, ZB)])
    plsc.subcore_barrier()

    base_e = s * ES
    c64 = c * HALF

    def issue_idx(e0, p):
        pltpu.async_copy(src_hbm.at[pl.ds(e0, EB)], srcb[p], sidx[p])
        pltpu.async_copy(dst_hbm.at[pl.ds(e0, EB)], dstb[p], sidx[p])

    def wait_idx(p):
        pltpu.make_async_copy(src_hbm.at[pl.ds(0, EB)], srcb[p], sidx[p]).wait()
        pltpu.make_async_copy(dst_hbm.at[pl.ds(0, EB)], dstb[p], sidx[p]).wait()

    def issue_dat(e0, p):
        pltpu.async_copy(xin_hbm.at[srcb[p]], xsb[p], sdat[p])
        pltpu.async_copy(ea_hbm.at[pl.ds(e0, EB)], eab[p], sdat[p])

    def wait_dat(p):
        pltpu.make_async_copy(xin_hbm.at[srcb[p]], xsb[p], sdat[p]).wait()
        pltpu.make_async_copy(ea_hbm.at[pl.ds(0, EB)], eab[p], sdat[p]).wait()

    def step(j, p):
        q = 1 - p

        @pl.when(j + 1 < NCHUNK)
        def _():
            wait_idx(q)
            issue_dat(base_e + (j + 1) * EB, q)

        wait_dat(p)

        @plsc.parallel_loop(0, EB, unroll=4)
        def _edge(e):
            for qq in range(HALF // 16):
                xq = xsb[p][e, pl.ds(c64 + 16 * qq, 16)]
                aq = eab[p][e, pl.ds(c64 + 16 * qq, 16)]
                msg = jnp.maximum(xq + aq, 0.0) + 1e-7
                pw = jnp.exp(msg * tv - uv)
                valb[e, pl.ds(16 * qq, 16)] = msg * pw
                valb[e, pl.ds(HALF + 16 * qq, 16)] = pw

        pltpu.sync_copy(valb, acc_sh.at[dstb[p]], add=True)

        @pl.when(j + 2 < NCHUNK)
        def _():
            issue_idx(base_e + (j + 2) * EB, p)

    issue_idx(base_e, 0)
    issue_idx(base_e + EB, 1)
    wait_idx(0)
    issue_dat(base_e, 0)

    def _pair(i, _):
        step(2 * i, 0)
        step(2 * i + 1, 1)
        return 0

    lax.fori_loop(0, NPAIR, _pair, 0)
    plsc.subcore_barrier()

    cN = c * N
    for k in range(NRROUND):
        cid = s + NS * k

        @pl.when(cid < NRCHUNK)
        def _():
            r0 = cid * RB
            pltpu.sync_copy(acc_sh.at[pl.ds(r0, RB)],
                            acc_hbm.at[pl.ds(cN + r0, RB)])


_sc_edge = pl.kernel(
    _sc_edge_body,
    out_type=jax.ShapeDtypeStruct((NC * N, 2 * HALF), _f32),
    mesh=plsc.VectorSubcoreMesh(core_axis_name="c", subcore_axis_name="s"),
    scratch_types=[
        pltpu.VMEM_SHARED((N, 2 * HALF), _f32),
        pltpu.VMEM((EB,), jnp.int32),
        pltpu.VMEM((EB,), jnp.int32),
        pltpu.VMEM((EB,), jnp.int32),
        pltpu.VMEM((EB,), jnp.int32),
        pltpu.VMEM((EB, H), _f32),
        pltpu.VMEM((EB, H), _f32),
        pltpu.VMEM((EB, H), _f32),
        pltpu.VMEM((EB, H), _f32),
        pltpu.VMEM((EB, 2 * HALF), _f32),
        pltpu.VMEM((16,), _f32),
        pltpu.VMEM((16,), _f32),
        pltpu.SemaphoreType.DMA,
        pltpu.SemaphoreType.DMA,
        pltpu.SemaphoreType.DMA,
        pltpu.SemaphoreType.DMA,
    ],
)


def _acc_max(mx_ref, m):
    i = pl.program_id(0)

    @pl.when(i == 0)
    def _():
        mx_ref[0, 0] = m

    @pl.when(i > 0)
    def _():
        mx_ref[0, 0] = jnp.maximum(mx_ref[0, 0], m)


def _k_edge_enc(ea_ref, w_ref, b_ref, out_ref, mx_ref):
    ea = jnp.dot(ea_ref[...], w_ref[...], preferred_element_type=_f32) + b_ref[...]
    out_ref[...] = ea
    _acc_max(mx_ref, jnp.max(ea))


def _k_node_enc(x_ref, w_ref, b_ref, out_ref, mx_ref):
    y = jnp.dot(x_ref[...], w_ref[...], preferred_element_type=_f32) + b_ref[...]
    out_ref[...] = y
    _acc_max(mx_ref, jnp.max(y))


def _ln_rows(x, g, b):
    mu = jnp.mean(x, axis=-1, keepdims=True)
    xc = x - mu
    var = jnp.mean(xc * xc, axis=-1, keepdims=True)
    return xc * jax.lax.rsqrt(var + 1e-5) * g + b


def _k_norm_act(x_ref, g_ref, b_ref, out_ref, mx_ref):
    y = jnp.maximum(_ln_rows(x_ref[...], g_ref[...], b_ref[...]), 0.0)
    out_ref[...] = y
    _acc_max(mx_ref, jnp.max(y))


def _k_mlp(xin_ref, acc_ref, xprev_ref, w1_ref, b1_ref, g1_ref, be1_ref,
           w2_ref, b2_ref, out_ref):
    a0 = acc_ref[0]
    a1 = acc_ref[1]
    lo = jnp.where(a0[:, HALF:] > 0.0, a0[:, :HALF] / a0[:, HALF:], 0.0)
    hi = jnp.where(a1[:, HALF:] > 0.0, a1[:, :HALF] / a1[:, HALF:], 0.0)
    h = xin_ref[...] + jnp.concatenate([lo, hi], axis=-1)
    z = jnp.dot(h, w1_ref[...], preferred_element_type=_f32) + b1_ref[...]
    z = jnp.maximum(_ln_rows(z, g1_ref[...], be1_ref[...]), 0.0)
    z = jnp.dot(z, w2_ref[...], preferred_element_type=_f32) + b2_ref[...]
    out_ref[...] = xprev_ref[...] + z


def _k_final(x_ref, g_ref, b_ref, w_ref, bo_ref, out_ref):
    y = jnp.maximum(_ln_rows(x_ref[...], g_ref[...], b_ref[...]), 0.0)
    out_ref[...] = jnp.dot(y, w_ref[...], preferred_element_type=_f32) + bo_ref[...]


def _full(shape):
    nd = len(shape)
    return pl.BlockSpec(shape, lambda i, _nd=nd: (0,) * _nd)


def _enc_max_call(body, x, w, b, bn):
    n = x.shape[0]
    return pl.pallas_call(
        body,
        grid=(n // bn,),
        in_specs=[
            pl.BlockSpec((bn, x.shape[1]), lambda i: (i, 0)),
            _full(w.shape),
            _full(b.shape),
        ],
        out_specs=[
            pl.BlockSpec((bn, H), lambda i: (i, 0)),
            pl.BlockSpec(memory_space=pltpu.SMEM),
        ],
        out_shape=[
            jax.ShapeDtypeStruct((n, H), _f32),
            jax.ShapeDtypeStruct((1, 1), _f32),
        ],
    )(x, w, b)


BN = 2000
BE_ENC = 4000


def kernel(x, edge_index, edge_attr, params):
    src = edge_index[0]
    dst = edge_index[1]

    We, be = params['edge_enc']
    ea, mx_ea = _enc_max_call(_k_edge_enc, edge_attr, We, be.reshape(1, H),
                              BE_ENC)

    Wn, bn_ = params['node_enc']
    xin, mx_x = _enc_max_call(_k_node_enc, x, Wn, bn_.reshape(1, H), BN)

    mlp_call = pl.pallas_call(
        _k_mlp,
        grid=(N // BN,),
        in_specs=[
            pl.BlockSpec((BN, H), lambda i: (i, 0)),
            pl.BlockSpec((2, BN, H), lambda i: (0, i, 0)),
            pl.BlockSpec((BN, H), lambda i: (i, 0)),
            _full((H, 2 * H)), _full((1, 2 * H)), _full((1, 2 * H)),
            _full((1, 2 * H)), _full((2 * H, H)), _full((1, H)),
        ],
        out_specs=pl.BlockSpec((BN, H), lambda i: (i, 0)),
        out_shape=jax.ShapeDtypeStruct((N, H), _f32),
    )

    x_run = jnp.zeros((N, H), _f32)
    for li, lp in enumerate(params['layers']):
        if li > 0:
            g, bb = lp['norm']
            xin, mx_x = _enc_max_call(
                _k_norm_act, x_run, g.reshape(1, H), bb.reshape(1, H), BN)
        t = lp['t']
        u = t * (jnp.maximum(mx_x[0, 0] + mx_ea[0, 0], 0.0) + 1e-7)
        t16 = jnp.broadcast_to(t.astype(_f32), (16,))
        u16 = jnp.broadcast_to(u.astype(_f32), (16,))
        acc_flat = _sc_edge(xin, ea, src, dst, t16, u16)
        acc2 = acc_flat.reshape(NC, N, 2 * HALF)
        W1, b1 = lp['mlp_w1']
        g1, be1 = lp['mlp_ln']
        W2, b2 = lp['mlp_w2']
        x_run = mlp_call(xin, acc2, x_run, W1, b1.reshape(1, 2 * H),
                         g1.reshape(1, 2 * H), be1.reshape(1, 2 * H),
                         W2, b2.reshape(1, H))

    g0, b0 = params['layers'][0]['norm']
    Wo, bo = params['lin_out']
    out = pl.pallas_call(
        _k_final,
        grid=(N // BN,),
        in_specs=[
            pl.BlockSpec((BN, H), lambda i: (i, 0)),
            _full((1, H)), _full((1, H)), _full((H, H)), _full((1, H)),
        ],
        out_specs=pl.BlockSpec((BN, H), lambda i: (i, 0)),
        out_shape=jax.ShapeDtypeStruct((N, H), _f32),
    )(x_run, g0.reshape(1, H), b0.reshape(1, H), Wo, bo.reshape(1, H))
    return out


# speedup vs baseline: 9.1706x; 1.2399x over previous
import jax
import jax.numpy as jnp
from jax import lax
from jax.experimental import pallas as pl
from jax.experimental.pallas import tpu as pltpu
from jax.experimental.pallas import tpu_sc as plsc

N = 10000
E = 320000
H = 128
HALF = 64
NC = 2
NS = 16
EB = 40
ES = E // NS
NCHUNK = ES // EB
NPAIR = NCHUNK // 2
ZB = 40
NZCHUNK = N // ZB
NZROUND = (NZCHUNK + NS - 1) // NS
RB = 200
NRCHUNK = N // RB
NRROUND = (NRCHUNK + NS - 1) // NS

_f32 = jnp.float32


def _sc_edge_body(xin_hbm, ea_hbm, src_hbm, dst_hbm, t_hbm, u_hbm, acc_hbm,
                  acc_sh, srcb0, srcb1, dstb0, dstb1, xsb0, xsb1, eab0, eab1,
                  valb0, valb1, tb, ub, ssrc0, ssrc1, sdat0, sdat1,
                  sdst0, sdst1, sscat0, sscat1):
    srcb = (srcb0, srcb1)
    dstb = (dstb0, dstb1)
    xsb = (xsb0, xsb1)
    eab = (eab0, eab1)
    valb = (valb0, valb1)
    ssrc = (ssrc0, ssrc1)
    sdat = (sdat0, sdat1)
    sdst = (sdst0, sdst1)
    sscat = (sscat0, sscat1)
    c = lax.axis_index("c")
    s = lax.axis_index("s")

    pltpu.sync_copy(t_hbm, tb)
    pltpu.sync_copy(u_hbm, ub)
    tv = tb[...]
    uv = ub[...]

    zero16 = jnp.zeros((16,), _f32)

    def _zrow(r, _):
        for q in range(2 * HALF // 16):
            valb0[r, pl.ds(16 * q, 16)] = zero16
        return 0

    lax.fori_loop(0, ZB, _zrow, 0)
    for k in range(NZROUND):
        cid = s + NS * k

        @pl.when(cid < NZCHUNK)
        def _():
            pltpu.sync_copy(valb0, acc_sh.at[pl.ds(cid * ZB, ZB)])
    plsc.subcore_barrier()

    base_e = s * ES
    c64 = c * HALF

    def issue_src(e0, p):
        pltpu.async_copy(src_hbm.at[pl.ds(e0, EB)], srcb[p], ssrc[p])

    def wait_src(p):
        pltpu.make_async_copy(src_hbm.at[pl.ds(0, EB)], srcb[p], ssrc[p]).wait()

    def issue_dat(e0, p):
        pltpu.async_copy(xin_hbm.at[srcb[p]], xsb[p], sdat[p])
        pltpu.async_copy(ea_hbm.at[pl.ds(e0, EB)], eab[p], sdat[p])

    def wait_dat(p):
        pltpu.make_async_copy(xin_hbm.at[srcb[p]], xsb[p], sdat[p]).wait()
        pltpu.make_async_copy(ea_hbm.at[pl.ds(0, EB)], eab[p], sdat[p]).wait()

    def wait_scat(p):
        pltpu.make_async_copy(valb[p], acc_sh.at[dstb[p]], sscat[p]).wait()

    def step(j, p):
        q = 1 - p

        @pl.when(j + 1 < NCHUNK)
        def _():
            wait_src(q)
            issue_dat(base_e + (j + 1) * EB, q)

        wait_dat(p)

        @pl.when(j + 2 < NCHUNK)
        def _():
            issue_src(base_e + (j + 2) * EB, p)

        @pl.when(j >= 2)
        def _():
            wait_scat(p)

        pltpu.async_copy(dst_hbm.at[pl.ds(base_e + j * EB, EB)], dstb[p],
                         sdst[p])

        @plsc.parallel_loop(0, EB, unroll=4)
        def _edge(e):
            for qq in range(HALF // 16):
                xq = xsb[p][e, pl.ds(c64 + 16 * qq, 16)]
                aq = eab[p][e, pl.ds(c64 + 16 * qq, 16)]
                msg = jnp.maximum(xq + aq, 0.0) + 1e-7
                pw = jnp.exp(msg * tv - uv)
                valb[p][e, pl.ds(16 * qq, 16)] = msg * pw
                valb[p][e, pl.ds(HALF + 16 * qq, 16)] = pw

        pltpu.make_async_copy(dst_hbm.at[pl.ds(0, EB)], dstb[p], sdst[p]).wait()
        pltpu.async_copy(valb[p], acc_sh.at[dstb[p]], sscat[p], add=True)

    issue_src(base_e, 0)
    issue_src(base_e + EB, 1)
    wait_src(0)
    issue_dat(base_e, 0)

    def _pair(i, _):
        step(2 * i, 0)
        step(2 * i + 1, 1)
        return 0

    lax.fori_loop(0, NPAIR, _pair, 0)
    wait_scat(0)
    wait_scat(1)
    plsc.subcore_barrier()

    cN = c * N
    for k in range(NRROUND):
        cid = s + NS * k

        @pl.when(cid < NRCHUNK)
        def _():
            r0 = cid * RB
            pltpu.sync_copy(acc_sh.at[pl.ds(r0, RB)],
                            acc_hbm.at[pl.ds(cN + r0, RB)])


_sc_edge = pl.kernel(
    _sc_edge_body,
    out_type=jax.ShapeDtypeStruct((NC * N, 2 * HALF), _f32),
    mesh=plsc.VectorSubcoreMesh(core_axis_name="c", subcore_axis_name="s"),
    scratch_types=[
        pltpu.VMEM_SHARED((N, 2 * HALF), _f32),
        pltpu.VMEM((EB,), jnp.int32),
        pltpu.VMEM((EB,), jnp.int32),
        pltpu.VMEM((EB,), jnp.int32),
        pltpu.VMEM((EB,), jnp.int32),
        pltpu.VMEM((EB, H), _f32),
        pltpu.VMEM((EB, H), _f32),
        pltpu.VMEM((EB, H), _f32),
        pltpu.VMEM((EB, H), _f32),
        pltpu.VMEM((EB, 2 * HALF), _f32),
        pltpu.VMEM((EB, 2 * HALF), _f32),
        pltpu.VMEM((16,), _f32),
        pltpu.VMEM((16,), _f32),
        pltpu.SemaphoreType.DMA,
        pltpu.SemaphoreType.DMA,
        pltpu.SemaphoreType.DMA,
        pltpu.SemaphoreType.DMA,
        pltpu.SemaphoreType.DMA,
        pltpu.SemaphoreType.DMA,
        pltpu.SemaphoreType.DMA,
        pltpu.SemaphoreType.DMA,
    ],
)


def _acc_max(mx_ref, m):
    i = pl.program_id(0)

    @pl.when(i == 0)
    def _():
        mx_ref[0, 0] = m

    @pl.when(i > 0)
    def _():
        mx_ref[0, 0] = jnp.maximum(mx_ref[0, 0], m)


def _k_edge_enc(ea_ref, w_ref, b_ref, out_ref, mx_ref):
    ea = jnp.dot(ea_ref[...], w_ref[...], preferred_element_type=_f32) + b_ref[...]
    out_ref[...] = ea
    _acc_max(mx_ref, jnp.max(ea))


def _k_node_enc(x_ref, w_ref, b_ref, out_ref, mx_ref):
    y = jnp.dot(x_ref[...], w_ref[...], preferred_element_type=_f32) + b_ref[...]
    out_ref[...] = y
    _acc_max(mx_ref, jnp.max(y))


def _ln_rows(x, g, b):
    mu = jnp.mean(x, axis=-1, keepdims=True)
    xc = x - mu
    var = jnp.mean(xc * xc, axis=-1, keepdims=True)
    return xc * jax.lax.rsqrt(var + 1e-5) * g + b


def _k_norm_act(x_ref, g_ref, b_ref, out_ref, mx_ref):
    y = jnp.maximum(_ln_rows(x_ref[...], g_ref[...], b_ref[...]), 0.0)
    out_ref[...] = y
    _acc_max(mx_ref, jnp.max(y))


def _k_mlp(xin_ref, acc_ref, xprev_ref, w1_ref, b1_ref, g1_ref, be1_ref,
           w2_ref, b2_ref, out_ref):
    a0 = acc_ref[0]
    a1 = acc_ref[1]
    lo = jnp.where(a0[:, HALF:] > 0.0, a0[:, :HALF] / a0[:, HALF:], 0.0)
    hi = jnp.where(a1[:, HALF:] > 0.0, a1[:, :HALF] / a1[:, HALF:], 0.0)
    h = xin_ref[...] + jnp.concatenate([lo, hi], axis=-1)
    z = jnp.dot(h, w1_ref[...], preferred_element_type=_f32) + b1_ref[...]
    z = jnp.maximum(_ln_rows(z, g1_ref[...], be1_ref[...]), 0.0)
    z = jnp.dot(z, w2_ref[...], preferred_element_type=_f32) + b2_ref[...]
    out_ref[...] = xprev_ref[...] + z


def _k_final(x_ref, g_ref, b_ref, w_ref, bo_ref, out_ref):
    y = jnp.maximum(_ln_rows(x_ref[...], g_ref[...], b_ref[...]), 0.0)
    out_ref[...] = jnp.dot(y, w_ref[...], preferred_element_type=_f32) + bo_ref[...]


def _full(shape):
    nd = len(shape)
    return pl.BlockSpec(shape, lambda i, _nd=nd: (0,) * _nd)


def _enc_max_call(body, x, w, b, bn):
    n = x.shape[0]
    return pl.pallas_call(
        body,
        grid=(n // bn,),
        in_specs=[
            pl.BlockSpec((bn, x.shape[1]), lambda i: (i, 0)),
            _full(w.shape),
            _full(b.shape),
        ],
        out_specs=[
            pl.BlockSpec((bn, H), lambda i: (i, 0)),
            pl.BlockSpec(memory_space=pltpu.SMEM),
        ],
        out_shape=[
            jax.ShapeDtypeStruct((n, H), _f32),
            jax.ShapeDtypeStruct((1, 1), _f32),
        ],
    )(x, w, b)


BN = 2000
BE_ENC = 4000


def kernel(x, edge_index, edge_attr, params):
    src = edge_index[0]
    dst = edge_index[1]

    We, be = params['edge_enc']
    ea, mx_ea = _enc_max_call(_k_edge_enc, edge_attr, We, be.reshape(1, H),
                              BE_ENC)

    Wn, bn_ = params['node_enc']
    xin, mx_x = _enc_max_call(_k_node_enc, x, Wn, bn_.reshape(1, H), BN)

    mlp_call = pl.pallas_call(
        _k_mlp,
        grid=(N // BN,),
        in_specs=[
            pl.BlockSpec((BN, H), lambda i: (i, 0)),
            pl.BlockSpec((2, BN, H), lambda i: (0, i, 0)),
            pl.BlockSpec((BN, H), lambda i: (i, 0)),
            _full((H, 2 * H)), _full((1, 2 * H)), _full((1, 2 * H)),
            _full((1, 2 * H)), _full((2 * H, H)), _full((1, H)),
        ],
        out_specs=pl.BlockSpec((BN, H), lambda i: (i, 0)),
        out_shape=jax.ShapeDtypeStruct((N, H), _f32),
    )

    x_run = jnp.zeros((N, H), _f32)
    for li, lp in enumerate(params['layers']):
        if li > 0:
            g, bb = lp['norm']
            xin, mx_x = _enc_max_call(
                _k_norm_act, x_run, g.reshape(1, H), bb.reshape(1, H), BN)
        t = lp['t']
        u = t * (jnp.maximum(mx_x[0, 0] + mx_ea[0, 0], 0.0) + 1e-7)
        t16 = jnp.broadcast_to(t.astype(_f32), (16,))
        u16 = jnp.broadcast_to(u.astype(_f32), (16,))
        acc_flat = _sc_edge(xin, ea, src, dst, t16, u16)
        acc2 = acc_flat.reshape(NC, N, 2 * HALF)
        W1, b1 = lp['mlp_w1']
        g1, be1 = lp['mlp_ln']
        W2, b2 = lp['mlp_w2']
        x_run = mlp_call(xin, acc2, x_run, W1, b1.reshape(1, 2 * H),
                         g1.reshape(1, 2 * H), be1.reshape(1, 2 * H),
                         W2, b2.reshape(1, H))

    g0, b0 = params['layers'][0]['norm']
    Wo, bo = params['lin_out']
    out = pl.pallas_call(
        _k_final,
        grid=(N // BN,),
        in_specs=[
            pl.BlockSpec((BN, H), lambda i: (i, 0)),
            _full((1, H)), _full((1, H)), _full((H, H)), _full((1, H)),
        ],
        out_specs=pl.BlockSpec((BN, H), lambda i: (i, 0)),
        out_shape=jax.ShapeDtypeStruct((N, H), _f32),
    )(x_run, g0.reshape(1, H), b0.reshape(1, H), Wo, bo.reshape(1, H))
    return out
